# baseline (speedup 1.0000x reference)
import jax
import jax.numpy as jnp
from jax.experimental import pallas as pl
from jax.experimental.pallas import tpu as pltpu
from jax.experimental.pallas import tpu_sc as plsc

_WINDOW = 128


def kernel(x, weight):
    b, s = x.shape
    n = b * s
    nv, d = weight.shape

    zf = jax.lax.optimization_barrier(jnp.float32(0))
    zi = jax.lax.optimization_barrier(jnp.int32(0))
    idx = x.reshape(n // _WINDOW, _WINDOW).astype(jnp.int32) + zi
    w_in = weight + zf

    mesh = plsc.VectorSubcoreMesh(core_axis_name="core", subcore_axis_name="subcore")

    @pl.kernel(
        out_type=jax.ShapeDtypeStruct((n, d), jnp.float32),
        mesh=mesh,
        compiler_params=pltpu.CompilerParams(use_tc_tiling_on_sc=False),
    )
    def gather_kernel(w_hbm, i_hbm, o_hbm):
        def body(i_vmem, o_vmem):
            pltpu.sync_copy(w_hbm.at[i_vmem.at[0]], o_vmem)

        pltpu.emit_pipeline(
            body,
            grid=(n // _WINDOW,),
            in_specs=[pl.BlockSpec((1, _WINDOW), index_map=lambda i: (i, 0))],
            out_specs=[pl.BlockSpec((_WINDOW, d), index_map=lambda i: (i, 0))],
            core_axis_name=("core", "subcore"),
            dimension_semantics=(pltpu.PARALLEL,),
        )(i_hbm, o_hbm)

    out = gather_kernel(w_in, idx)
    return out.reshape(b, s, d) + zf


# speedup vs baseline: 1.3315x; 1.3315x over previous
import jax
import jax.numpy as jnp
from jax.experimental import pallas as pl
from jax.experimental.pallas import tpu as pltpu
from jax.experimental.pallas import tpu_sc as plsc

_WINDOW = 128


def kernel(x, weight):
    b, s = x.shape
    n = b * s
    nv, d = weight.shape

    idx = x.reshape(n).astype(jnp.int32)

    mesh = plsc.VectorSubcoreMesh(core_axis_name="core", subcore_axis_name="subcore")

    @pl.kernel(
        out_type=jax.ShapeDtypeStruct((n, d), jnp.float32),
        mesh=mesh,
        compiler_params=pltpu.CompilerParams(use_tc_tiling_on_sc=False),
    )
    def gather_kernel(w_hbm, i_hbm, o_hbm):
        def body(i_vmem, o_vmem):
            pltpu.sync_copy(w_hbm.at[i_vmem], o_vmem)

        pltpu.emit_pipeline(
            body,
            grid=(n // _WINDOW,),
            in_specs=[pl.BlockSpec((_WINDOW,), index_map=lambda i: (i,))],
            out_specs=[pl.BlockSpec((_WINDOW, d), index_map=lambda i: (i, 0))],
            core_axis_name=("core", "subcore"),
            dimension_semantics=(pltpu.PARALLEL,),
        )(i_hbm, o_hbm)

    out = gather_kernel(weight, idx)
    return out.reshape(b, s, d)


# speedup vs baseline: 1.6950x; 1.2730x over previous
import dataclasses

import jax
import jax.numpy as jnp
from jax.experimental import pallas as pl
from jax.experimental.pallas import tpu as pltpu
from jax.experimental.pallas import tpu_sc as plsc

_W = 128
_L = 16

_CP = pltpu.CompilerParams(use_tc_tiling_on_sc=False)
if "needs_layout_passes" in pltpu.CompilerParams.__dataclass_fields__:
    _CP = dataclasses.replace(_CP, needs_layout_passes=False)


def kernel(x, weight):
    b, s = x.shape
    nv, d = weight.shape

    idx_t = x.T.astype(jnp.int32)

    mesh = plsc.VectorSubcoreMesh(core_axis_name="core", subcore_axis_name="subcore")

    @pl.kernel(
        out_type=jax.ShapeDtypeStruct((s, d, b), jnp.float32),
        mesh=mesh,
        scratch_types=[pltpu.VMEM((_W, d), jnp.float32)],
        compiler_params=_CP,
    )
    def gather_kernel(w_hbm, i_hbm, o_hbm, scratch):
        def body(i_vm, o_vm):
            pltpu.sync_copy(w_hbm.at[i_vm.at[0]], scratch)
            iota = jax.lax.iota(jnp.int32, _L)
            for k in range(_W // _L):
                rows = iota + (_L * k)
                for dd in range(d):
                    cols = jnp.full((_L,), dd, jnp.int32)
                    o_vm[0, dd, pl.ds(_L * k, _L)] = plsc.load_gather(
                        scratch, [rows, cols]
                    )

        pltpu.emit_pipeline(
            body,
            grid=(s, b // _W),
            in_specs=[pl.BlockSpec((1, _W), index_map=lambda si, ui: (si, ui))],
            out_specs=[pl.BlockSpec((1, d, _W), index_map=lambda si, ui: (si, 0, ui))],
            core_axis_name=("core", "subcore"),
            dimension_semantics=(pltpu.PARALLEL, pltpu.PARALLEL),
        )(i_hbm, o_hbm)

    out_t = gather_kernel(weight, idx_t)
    return jnp.transpose(out_t, (2, 0, 1))


# speedup vs baseline: 2.0321x; 1.1989x over previous
import dataclasses

import jax
import jax.numpy as jnp
from jax.experimental import pallas as pl
from jax.experimental.pallas import tpu as pltpu
from jax.experimental.pallas import tpu_sc as plsc

_W = 128
_L = 16

_CP = pltpu.CompilerParams(use_tc_tiling_on_sc=False)
if "needs_layout_passes" in pltpu.CompilerParams.__dataclass_fields__:
    _CP = dataclasses.replace(_CP, needs_layout_passes=False)


def kernel(x, weight):
    b, s = x.shape
    nv, d = weight.shape

    idx_t = x.T.astype(jnp.int32)

    mesh = plsc.VectorSubcoreMesh(core_axis_name="core", subcore_axis_name="subcore")

    @pl.kernel(
        out_type=jax.ShapeDtypeStruct((s, d, b), jnp.float32),
        mesh=mesh,
        scratch_types=[pltpu.VMEM((_W, d), jnp.float32)],
        compiler_params=_CP,
    )
    def gather_kernel(w_hbm, i_hbm, o_hbm, scratch):
        def body(i_vm, o_vm):
            pltpu.sync_copy(w_hbm.at[i_vm.at[0]], scratch)
            iota = jax.lax.iota(jnp.int32, _L)

            @plsc.parallel_loop(0, _W, step=_L)
            def _(c):
                rows = iota + c
                for dd in range(d):
                    cols = jnp.full((_L,), dd, jnp.int32)
                    o_vm[0, dd, pl.ds(c, _L)] = plsc.load_gather(
                        scratch, [rows, cols]
                    )

        pltpu.emit_pipeline(
            body,
            grid=(s, b // _W),
            in_specs=[pl.BlockSpec((1, _W), index_map=lambda si, ui: (si, ui))],
            out_specs=[pl.BlockSpec((1, d, _W), index_map=lambda si, ui: (si, 0, ui))],
            core_axis_name=("core", "subcore"),
            dimension_semantics=(pltpu.PARALLEL, pltpu.PARALLEL),
        )(i_hbm, o_hbm)

    out_t = gather_kernel(weight, idx_t)
    return jnp.transpose(out_t, (2, 0, 1))


# speedup vs baseline: 2.2012x; 1.0832x over previous
import dataclasses

import jax
import jax.numpy as jnp
from jax.experimental import pallas as pl
from jax.experimental.pallas import tpu as pltpu
from jax.experimental.pallas import tpu_sc as plsc

_W = 256
_L = 16

_CP = pltpu.CompilerParams(use_tc_tiling_on_sc=False)
if "needs_layout_passes" in pltpu.CompilerParams.__dataclass_fields__:
    _CP = dataclasses.replace(_CP, needs_layout_passes=False)


def kernel(x, weight):
    b, s = x.shape
    nv, d = weight.shape

    idx_t = x.T.astype(jnp.int32)

    mesh = plsc.VectorSubcoreMesh(core_axis_name="core", subcore_axis_name="subcore")

    @pl.kernel(
        out_type=jax.ShapeDtypeStruct((s, d, b), jnp.float32),
        mesh=mesh,
        scratch_types=[pltpu.VMEM((_W, d), jnp.float32)],
        compiler_params=_CP,
    )
    def gather_kernel(w_hbm, i_hbm, o_hbm, scratch):
        def body(i_vm, o_vm):
            pltpu.sync_copy(w_hbm.at[i_vm.at[0]], scratch)
            iota = jax.lax.iota(jnp.int32, _L)

            @plsc.parallel_loop(0, _W, step=_L)
            def _(c):
                rows = iota + c
                for dd in range(d):
                    cols = jnp.full((_L,), dd, jnp.int32)
                    o_vm[0, dd, pl.ds(c, _L)] = plsc.load_gather(
                        scratch, [rows, cols]
                    )

        pltpu.emit_pipeline(
            body,
            grid=(s, b // _W),
            in_specs=[pl.BlockSpec((1, _W), index_map=lambda si, ui: (si, ui))],
            out_specs=[pl.BlockSpec((1, d, _W), index_map=lambda si, ui: (si, 0, ui))],
            core_axis_name=("core", "subcore"),
            dimension_semantics=(pltpu.PARALLEL, pltpu.PARALLEL),
        )(i_hbm, o_hbm)

    out_t = gather_kernel(weight, idx_t)
    return jnp.transpose(out_t, (2, 0, 1))


# speedup vs baseline: 2.2702x; 1.0313x over previous
import dataclasses

import jax
import jax.numpy as jnp
from jax.experimental import pallas as pl
from jax.experimental.pallas import tpu as pltpu
from jax.experimental.pallas import tpu_sc as plsc

_W = 512
_L = 16

_CP = pltpu.CompilerParams(use_tc_tiling_on_sc=False)
if "needs_layout_passes" in pltpu.CompilerParams.__dataclass_fields__:
    _CP = dataclasses.replace(_CP, needs_layout_passes=False)


def kernel(x, weight):
    b, s = x.shape
    nv, d = weight.shape

    idx_t = x.T.astype(jnp.int32)

    mesh = plsc.VectorSubcoreMesh(core_axis_name="core", subcore_axis_name="subcore")

    @pl.kernel(
        out_type=jax.ShapeDtypeStruct((s, d, b), jnp.float32),
        mesh=mesh,
        scratch_types=[pltpu.VMEM((_W, d), jnp.float32)],
        compiler_params=_CP,
    )
    def gather_kernel(w_hbm, i_hbm, o_hbm, scratch):
        def body(i_vm, o_vm):
            pltpu.sync_copy(w_hbm.at[i_vm.at[0]], scratch)
            iota = jax.lax.iota(jnp.int32, _L)

            @plsc.parallel_loop(0, _W, step=_L)
            def _(c):
                rows = iota + c
                for dd in range(d):
                    cols = jnp.full((_L,), dd, jnp.int32)
                    o_vm[0, dd, pl.ds(c, _L)] = plsc.load_gather(
                        scratch, [rows, cols]
                    )

        pltpu.emit_pipeline(
            body,
            grid=(s, b // _W),
            in_specs=[pl.BlockSpec((1, _W), index_map=lambda si, ui: (si, ui))],
            out_specs=[pl.BlockSpec((1, d, _W), index_map=lambda si, ui: (si, 0, ui))],
            core_axis_name=("core", "subcore"),
            dimension_semantics=(pltpu.PARALLEL, pltpu.PARALLEL),
        )(i_hbm, o_hbm)

    out_t = gather_kernel(weight, idx_t)
    return jnp.transpose(out_t, (2, 0, 1))
